# baseline (speedup 1.0000x reference)
import functools

import jax
import jax.numpy as jnp
from jax import lax
from jax.experimental import pallas as pl
from jax.experimental.pallas import tpu as pltpu
from jax.experimental.pallas import tpu_sc as plsc

_info = plsc.get_sparse_core_info()
_NC, _NS, _L = _info.num_cores, _info.num_subcores, _info.num_lanes
_NW = _NC * _NS

_BATCH = 16384
_BPW = _BATCH // _NW
_CHUNK = 64
_NCHUNK = _BPW // _CHUNK


@functools.partial(
    pl.kernel,
    out_type=jax.ShapeDtypeStruct((_NW, _NCHUNK, _CHUNK), jnp.float32),
    mesh=plsc.VectorSubcoreMesh(core_axis_name="c", subcore_axis_name="s"),
    scratch_types=[
        pltpu.VMEM((_NCHUNK, _CHUNK), jnp.int32),
        pltpu.VMEM((_NCHUNK, _CHUNK), jnp.int32),
        pltpu.VMEM((_NCHUNK, _CHUNK), jnp.int32),
        pltpu.VMEM((_NCHUNK, _CHUNK), jnp.float32),
        pltpu.VMEM((_NCHUNK, _CHUNK), jnp.float32),
        pltpu.VMEM((_L,), jnp.float32),
        pltpu.VMEM((_NCHUNK, _CHUNK), jnp.float32),
        pltpu.SemaphoreType.DMA,
        pltpu.SemaphoreType.DMA,
        [pltpu.SemaphoreType.DMA] * _NCHUNK,
        pltpu.SemaphoreType.DMA,
    ],
)
def _sc_body(uid_hbm, iid_hbm, r_hbm, u_hbm, i_hbm, z_hbm, out_hbm,
             uid_v, iid_v, r_v, uv, iv, z_v, ov, sem_ld, sem_ld2, sems, sem_st):
    wid = lax.axis_index("s") * _NC + lax.axis_index("c")

    idx_loads = [
        pltpu.async_copy(uid_hbm.at[wid], uid_v, sem_ld),
        pltpu.async_copy(iid_hbm.at[wid], iid_v, sem_ld),
    ]
    rz_loads = [
        pltpu.async_copy(r_hbm.at[wid], r_v, sem_ld2),
        pltpu.async_copy(z_hbm, z_v, sem_ld2),
    ]
    for cp in idx_loads:
        cp.wait()

    gathers = []
    for j in range(_NCHUNK):
        gathers.append((
            pltpu.async_copy(u_hbm.at[uid_v.at[j]], uv.at[j], sems[j]),
            pltpu.async_copy(i_hbm.at[iid_v.at[j]], iv.at[j], sems[j]),
        ))

    for cp in rz_loads:
        cp.wait()
    zvec = z_v[...]
    stores = []
    for j in range(_NCHUNK):
        for cp in gathers[j]:
            cp.wait()
        for c in range(_CHUNK // _L):
            sl = pl.ds(c * _L, _L)
            zr = zvec.at[r_v[j, sl]].get(mode="promise_in_bounds")
            ov[j, sl] = jnp.exp(uv[j, sl] + iv[j, sl] + zr)
        stores.append(pltpu.async_copy(ov.at[j], out_hbm.at[wid, j], sem_st))
    for cp in stores:
        cp.wait()


@jax.jit
def kernel(uid, iid, r, u, i, z):
    uid3 = uid.astype(jnp.int32).reshape(_NW, _NCHUNK, _CHUNK)
    iid3 = iid.astype(jnp.int32).reshape(_NW, _NCHUNK, _CHUNK)
    r3 = r.astype(jnp.int32).reshape(_NW, _NCHUNK, _CHUNK)
    z16 = jnp.zeros((_L,), jnp.float32).at[: z.shape[0]].set(z)
    out = _sc_body(uid3, iid3, r3, u, i, z16)
    return out.reshape(-1)


# speedup vs baseline: 1.2416x; 1.2416x over previous
import functools

import jax
import jax.numpy as jnp
from jax import lax
from jax.experimental import pallas as pl
from jax.experimental.pallas import tpu as pltpu
from jax.experimental.pallas import tpu_sc as plsc

_info = plsc.get_sparse_core_info()
_NC, _NS, _L = _info.num_cores, _info.num_subcores, _info.num_lanes
_NW = _NC * _NS

_BATCH = 16384
_BPW = _BATCH // _NW
_CHUNK = 128
_NCHUNK = _BPW // _CHUNK


@functools.partial(
    pl.kernel,
    out_type=jax.ShapeDtypeStruct((_NW, _NCHUNK, _CHUNK), jnp.float32),
    mesh=plsc.VectorSubcoreMesh(core_axis_name="c", subcore_axis_name="s"),
    scratch_types=[
        pltpu.VMEM((_NCHUNK, _CHUNK), jnp.int32),
        pltpu.VMEM((_NCHUNK, _CHUNK), jnp.int32),
        pltpu.VMEM((_NCHUNK, _CHUNK), jnp.int32),
        pltpu.VMEM((_NCHUNK, _CHUNK), jnp.float32),
        pltpu.VMEM((_NCHUNK, _CHUNK), jnp.float32),
        pltpu.VMEM((_L,), jnp.float32),
        pltpu.VMEM((_NCHUNK, _CHUNK), jnp.float32),
        pltpu.SemaphoreType.DMA,
        pltpu.SemaphoreType.DMA,
        [pltpu.SemaphoreType.DMA] * _NCHUNK,
        pltpu.SemaphoreType.DMA,
    ],
)
def _sc_body(uid_hbm, iid_hbm, r_hbm, u_hbm, i_hbm, z_hbm, out_hbm,
             uid_v, iid_v, r_v, uv, iv, z_v, ov, sem_ld, sem_ld2, sems, sem_st):
    wid = lax.axis_index("s") * _NC + lax.axis_index("c")

    idx_loads = [
        pltpu.async_copy(uid_hbm.at[wid], uid_v, sem_ld),
        pltpu.async_copy(iid_hbm.at[wid], iid_v, sem_ld),
    ]
    rz_loads = [
        pltpu.async_copy(r_hbm.at[wid], r_v, sem_ld2),
        pltpu.async_copy(z_hbm, z_v, sem_ld2),
    ]
    for cp in idx_loads:
        cp.wait()

    gathers = []
    for j in range(_NCHUNK):
        gathers.append((
            pltpu.async_copy(u_hbm.at[uid_v.at[j]], uv.at[j], sems[j]),
            pltpu.async_copy(i_hbm.at[iid_v.at[j]], iv.at[j], sems[j]),
        ))

    for cp in rz_loads:
        cp.wait()
    zvec = z_v[...]
    stores = []
    for j in range(_NCHUNK):
        for cp in gathers[j]:
            cp.wait()
        for c in range(_CHUNK // _L):
            sl = pl.ds(c * _L, _L)
            zr = zvec.at[r_v[j, sl]].get(mode="promise_in_bounds")
            ov[j, sl] = jnp.exp(uv[j, sl] + iv[j, sl] + zr)
    del stores
    pltpu.sync_copy(ov, out_hbm.at[wid])


@jax.jit
def kernel(uid, iid, r, u, i, z):
    uid3 = uid.astype(jnp.int32).reshape(_NW, _NCHUNK, _CHUNK)
    iid3 = iid.astype(jnp.int32).reshape(_NW, _NCHUNK, _CHUNK)
    r3 = r.astype(jnp.int32).reshape(_NW, _NCHUNK, _CHUNK)
    z16 = jnp.zeros((_L,), jnp.float32).at[: z.shape[0]].set(z)
    out = _sc_body(uid3, iid3, r3, u, i, z16)
    return out.reshape(-1)


# speedup vs baseline: 1.2508x; 1.0074x over previous
import functools

import jax
import jax.numpy as jnp
from jax import lax
from jax.experimental import pallas as pl
from jax.experimental.pallas import tpu as pltpu
from jax.experimental.pallas import tpu_sc as plsc

_info = plsc.get_sparse_core_info()
_NC, _NS, _L = _info.num_cores, _info.num_subcores, _info.num_lanes
_NW = _NC * _NS

_BATCH = 16384
_BPW = _BATCH // _NW
_CHUNK = 128
_NCHUNK = _BPW // _CHUNK


@functools.partial(
    pl.kernel,
    out_type=jax.ShapeDtypeStruct((_NW, _NCHUNK, _CHUNK), jnp.float32),
    mesh=plsc.VectorSubcoreMesh(core_axis_name="c", subcore_axis_name="s"),
    scratch_types=[
        pltpu.VMEM((_NCHUNK, _CHUNK), jnp.int32),
        pltpu.VMEM((_NCHUNK, _CHUNK), jnp.int32),
        pltpu.VMEM((_NCHUNK, _CHUNK), jnp.int32),
        pltpu.VMEM((_NCHUNK, _CHUNK), jnp.float32),
        pltpu.VMEM((_NCHUNK, _CHUNK), jnp.float32),
        pltpu.VMEM((_L,), jnp.float32),
        pltpu.VMEM((_NCHUNK, _CHUNK), jnp.float32),
        pltpu.SemaphoreType.DMA,
        pltpu.SemaphoreType.DMA,
        [pltpu.SemaphoreType.DMA] * _NCHUNK,
        pltpu.SemaphoreType.DMA,
    ],
)
def _sc_body(uid_hbm, iid_hbm, r_hbm, u_hbm, i_hbm, z_hbm, out_hbm,
             uid_v, iid_v, r_v, uv, iv, z_v, ov, sem_ld, sem_ld2, sems, sem_st):
    wid = lax.axis_index("s") * _NC + lax.axis_index("c")

    idx_loads = [
        pltpu.async_copy(uid_hbm.at[wid], uid_v, sem_ld),
        pltpu.async_copy(iid_hbm.at[wid], iid_v, sem_ld),
    ]
    rz_loads = [
        pltpu.async_copy(r_hbm.at[wid], r_v, sem_ld2),
        pltpu.async_copy(z_hbm, z_v, sem_ld2),
    ]
    for cp in idx_loads:
        cp.wait()

    gathers = []
    for j in range(_NCHUNK):
        gathers.append((
            pltpu.async_copy(u_hbm.at[uid_v.at[j]], uv.at[j], sems[j]),
            pltpu.async_copy(i_hbm.at[iid_v.at[j]], iv.at[j], sems[j]),
        ))

    for cp in rz_loads:
        cp.wait()
    zvec = z_v[...]
    stores = []
    for j in range(_NCHUNK):
        for cp in gathers[j]:
            cp.wait()
        for c in range(_CHUNK // _L):
            sl = pl.ds(c * _L, _L)
            zr = zvec.at[r_v[j, sl]].get(mode="promise_in_bounds")
            ov[j, sl] = jnp.exp(uv[j, sl] + iv[j, sl] + zr)
        stores.append(pltpu.async_copy(ov.at[j], out_hbm.at[wid, j], sem_st))
    for cp in stores:
        cp.wait()


@jax.jit
def kernel(uid, iid, r, u, i, z):
    uid3 = uid.astype(jnp.int32).reshape(_NW, _NCHUNK, _CHUNK)
    iid3 = iid.astype(jnp.int32).reshape(_NW, _NCHUNK, _CHUNK)
    r3 = r.astype(jnp.int32).reshape(_NW, _NCHUNK, _CHUNK)
    z16 = jnp.zeros((_L,), jnp.float32).at[: z.shape[0]].set(z)
    out = _sc_body(uid3, iid3, r3, u, i, z16)
    return out.reshape(-1)
